# baseline (speedup 1.0000x reference)
import functools

import jax
import jax.numpy as jnp
from jax import lax
from jax.experimental import pallas as pl
from jax.experimental.pallas import tpu as pltpu
from jax.experimental.pallas import tpu_sc as plsc

B = 16384
L = 50
NUM_USERS = 100000
USER_DIM = 15
MAX_TOKENS = 10000
TOPIC_DIM = 32

NC = 2
NS = 16
NW = NC * NS
RPW = B // NW
CH = 128
NCH = RPW // CH
GR = 16
NG = RPW // GR
IDX_C = 100
NJ = GR * L // IDX_C
UCHUNK = 128
UNJ = RPW // UCHUNK


def _sc_body(tid2d, ids_t, ttable, uids3d, utab16,
             uout, tout,
             idxv0, idxv1, gbuf0, gbuf1, cntv, zbuf, row0v, uidx, ubuf,
             sbuf0, sbuf1, shm, sem0, sem1, osem0, osem1, usem, tsem,
             isem0, isem1):
    wid = lax.axis_index("s") * NC + lax.axis_index("c")
    wbase = wid * RPW
    idxvs = (idxv0, idxv1)
    gbufs = (gbuf0, gbuf1)
    sbufs = (sbuf0, sbuf1)
    sems = (sem0, sem1)
    osems = (osem0, osem1)
    isems = (isem0, isem1)
    GBYTES = GR * L * TOPIC_DIM * 4
    OBYTES = GR * TOPIC_DIM * 4
    IBYTES = NJ * IDX_C * 4

    pltpu.sync_copy(ttable.at[0], row0v)
    r0a = row0v[0:16]
    r0b = row0v[16:32]

    pltpu.sync_copy(uids3d.at[wid], uidx)
    for j in range(UNJ):
        pltpu.async_copy(utab16.at[uidx.at[j]],
                         ubuf.at[pl.ds(j * UCHUNK, UCHUNK), :], usem)

    sid = lax.axis_index("s")

    @pl.when(sid == 0)
    def _():
        pltpu.async_copy(ttable, shm, tsem)

    def cnt_chunk(c, carry):
        cb = wbase + c * CH
        pltpu.sync_copy(ids_t.at[:, pl.ds(pl.multiple_of(cb, CH), CH)], cntv)

        def cnt_body(t, zs):
            return tuple(
                zs[k] + jnp.where(cntv[t, k * 16:(k + 1) * 16] == 0, 1.0, 0.0)
                for k in range(CH // 16))

        zs = lax.fori_loop(0, L, cnt_body,
                           tuple(jnp.zeros((16,), jnp.float32)
                                 for _ in range(CH // 16)),
                           unroll=2)
        for k in range(CH // 16):
            zbuf[c * (CH // 16) + k, :] = zs[k]
        return carry

    lax.fori_loop(0, NCH, cnt_chunk, 0)

    @pl.when(sid == 0)
    def _():
        pltpu.make_async_copy(ttable, shm, tsem).wait()

    plsc.subcore_barrier()

    def stage_and_fire(g, par):
        base = wbase + g * GR
        ioff = pl.multiple_of(base // 2, NJ)
        pltpu.sync_copy(tid2d.at[pl.ds(ioff, NJ), :], idxvs[par])
        for j in range(NJ):
            pltpu.async_copy(ttable.at[idxvs[par].at[j]],
                             gbufs[par].at[pl.ds(j * IDX_C, IDX_C), :],
                             sems[par])

    stage_and_fire(0, 0)
    stage_and_fire(1, 1)

    def pair_body(p, carry):
        for par in range(2):
            g = 2 * p + par
            base = wbase + g * GR
            gbuf = gbufs[par]
            sbuf = sbufs[par]

            pltpu.make_async_copy(ttable.at[pl.ds(0, GR * L), :], gbuf,
                                  sems[par]).wait()

            @pl.when(g + 2 < NG)
            def _():
                base2 = wbase + (g + 2) * GR
                ioff2 = pl.multiple_of(base2 // 2, NJ)
                pltpu.async_copy(tid2d.at[pl.ds(ioff2, NJ), :], idxvs[par],
                                 isems[par])

            @pl.when(p > 0)
            def _():
                pltpu.make_async_copy(
                    sbuf, tout.at[pl.ds(pl.multiple_of(base, GR), GR), :],
                    osems[par]).wait()

            zv = zbuf[g, :]
            denv = jnp.maximum(jnp.float32(L) - zv, 1.0)

            for r in range(GR):
                b = r * L

                def sum_body(t, acc):
                    a0, a1 = acc
                    return (a0 + gbuf[b + t, 0:16],
                            a1 + gbuf[b + t, 16:32])

                a0, a1 = lax.fori_loop(
                    0, L, sum_body,
                    (jnp.zeros((16,), jnp.float32),
                     jnp.zeros((16,), jnp.float32)),
                    unroll=5)
                nz = zv[r]
                den = denv[r]
                sbuf[r, 0:16] = (a0 - nz * r0a) / den
                sbuf[r, 16:32] = (a1 - nz * r0b) / den

            @pl.when(g + 2 < NG)
            def _():
                pltpu.make_async_copy(tid2d.at[pl.ds(0, NJ), :], idxvs[par],
                                      isems[par]).wait()
                for j in range(NJ):
                    pltpu.async_copy(ttable.at[idxvs[par].at[j]],
                                     gbuf.at[pl.ds(j * IDX_C, IDX_C), :],
                                     sems[par])

            pltpu.async_copy(
                sbuf, tout.at[pl.ds(pl.multiple_of(base, GR), GR), :],
                osems[par])
        return carry

    lax.fori_loop(0, NG // 2, pair_body, 0)

    for par in range(2):
        pltpu.make_async_copy(
            sbufs[par],
            tout.at[pl.ds(pl.multiple_of(wbase, GR), GR), :],
            osems[par]).wait()

    pltpu.make_async_copy(utab16.at[pl.ds(0, RPW), :], ubuf, usem).wait()
    pltpu.sync_copy(ubuf, uout.at[pl.ds(pl.multiple_of(wbase, CH), RPW), :])


@functools.partial(
    pl.kernel,
    out_type=(
        jax.ShapeDtypeStruct((B, 16), jnp.float32),
        jax.ShapeDtypeStruct((B, TOPIC_DIM), jnp.float32),
    ),
    mesh=plsc.VectorSubcoreMesh(core_axis_name="c", subcore_axis_name="s"),
    compiler_params=pltpu.CompilerParams(use_tc_tiling_on_sc=False),
    scratch_types=[
        pltpu.VMEM((NJ, IDX_C), jnp.int32),
        pltpu.VMEM((NJ, IDX_C), jnp.int32),
        pltpu.VMEM((GR * L, TOPIC_DIM), jnp.float32),
        pltpu.VMEM((GR * L, TOPIC_DIM), jnp.float32),
        pltpu.VMEM((L, CH), jnp.int32),
        pltpu.VMEM((NG, 16), jnp.float32),
        pltpu.VMEM((TOPIC_DIM,), jnp.float32),
        pltpu.VMEM((UNJ, UCHUNK), jnp.int32),
        pltpu.VMEM((RPW, 16), jnp.float32),
        pltpu.VMEM((GR, TOPIC_DIM), jnp.float32),
        pltpu.VMEM((GR, TOPIC_DIM), jnp.float32),
        pltpu.VMEM_SHARED((MAX_TOKENS, TOPIC_DIM), jnp.float32),
        pltpu.SemaphoreType.DMA,
        pltpu.SemaphoreType.DMA,
        pltpu.SemaphoreType.DMA,
        pltpu.SemaphoreType.DMA,
        pltpu.SemaphoreType.DMA,
        pltpu.SemaphoreType.DMA,
        pltpu.SemaphoreType.DMA,
        pltpu.SemaphoreType.DMA,
    ],
)
def _user_model_sc(tid2d, ids_t, ttable, uids3d, utab16, uout, tout,
                   idxv0, idxv1, gbuf0, gbuf1, cntv, zbuf, row0v, uidx, ubuf,
                   sbuf0, sbuf1, shm, sem0, sem1, osem0, osem1, usem, tsem,
                   isem0, isem1):
    _sc_body(tid2d, ids_t, ttable, uids3d, utab16, uout, tout,
             idxv0, idxv1, gbuf0, gbuf1, cntv, zbuf, row0v, uidx, ubuf,
             sbuf0, sbuf1, shm, sem0, sem1, osem0, osem1, usem, tsem,
             isem0, isem1)


def kernel(user_ids, topic_ids, user_table, topic_table):
    tid2d = topic_ids.reshape(B * L // IDX_C, IDX_C)
    ids_t = topic_ids.T
    uids3d = user_ids.reshape(NW, UNJ, UCHUNK)
    utab16 = jnp.pad(user_table, ((0, 0), (0, 1)))
    uout, tout = _user_model_sc(tid2d, ids_t, topic_table, uids3d, utab16)
    return jnp.concatenate([uout[:, :USER_DIM], tout], axis=1)


# speedup vs baseline: 1.0301x; 1.0301x over previous
import functools

import jax
import jax.numpy as jnp
from jax import lax
from jax.experimental import pallas as pl
from jax.experimental.pallas import tpu as pltpu
from jax.experimental.pallas import tpu_sc as plsc

B = 16384
L = 50
NUM_USERS = 100000
USER_DIM = 15
MAX_TOKENS = 10000
TOPIC_DIM = 32

NC = 2
NS = 16
NW = NC * NS
RPW = B // NW
CH = 128
NCH = RPW // CH
GR = 16
NG = RPW // GR
IDX_C = 100
NJ = GR * L // IDX_C
UCHUNK = 128
UNJ = RPW // UCHUNK


def _sc_body(tid2d, ids_t, ttable, uids3d, utab16,
             uout, tout,
             idxv0, idxv1, gbuf0, gbuf1, cntv, zbuf, row0v, uidx, ubuf,
             sbuf0, sbuf1, shm, sem0, sem1, osem0, osem1, usem, tsem,
             isem0, isem1):
    wid = lax.axis_index("s") * NC + lax.axis_index("c")
    wbase = wid * RPW
    idxvs = (idxv0, idxv1)
    gbufs = (gbuf0, gbuf1)
    sbufs = (sbuf0, sbuf1)
    sems = (sem0, sem1)
    osems = (osem0, osem1)
    isems = (isem0, isem1)
    GBYTES = GR * L * TOPIC_DIM * 4
    OBYTES = GR * TOPIC_DIM * 4
    IBYTES = NJ * IDX_C * 4

    pltpu.sync_copy(ttable.at[0], row0v)
    r0a = row0v[0:16]
    r0b = row0v[16:32]

    pltpu.sync_copy(uids3d.at[wid], uidx)
    for j in range(UNJ):
        pltpu.async_copy(utab16.at[uidx.at[j]],
                         ubuf.at[pl.ds(j * UCHUNK, UCHUNK), :], usem)

    sid = lax.axis_index("s")

    @pl.when(sid == 0)
    def _():
        pltpu.async_copy(ttable, shm, tsem)

    def cnt_chunk(c, carry):
        cb = wbase + c * CH
        pltpu.sync_copy(ids_t.at[:, pl.ds(pl.multiple_of(cb, CH), CH)], cntv)

        def cnt_body(t, zs):
            return tuple(
                zs[k] + jnp.where(cntv[t, k * 16:(k + 1) * 16] == 0, 1.0, 0.0)
                for k in range(CH // 16))

        zs = lax.fori_loop(0, L, cnt_body,
                           tuple(jnp.zeros((16,), jnp.float32)
                                 for _ in range(CH // 16)),
                           unroll=2)
        for k in range(CH // 16):
            zbuf[c * (CH // 16) + k, :] = zs[k]
        return carry

    lax.fori_loop(0, NCH, cnt_chunk, 0)

    @pl.when(sid == 0)
    def _():
        pltpu.make_async_copy(ttable, shm, tsem).wait()

    plsc.subcore_barrier()

    srcs = (shm, ttable)

    def stage_and_fire(g, par):
        base = wbase + g * GR
        ioff = pl.multiple_of(base // 2, NJ)
        pltpu.sync_copy(tid2d.at[pl.ds(ioff, NJ), :], idxvs[par])
        for j in range(NJ):
            pltpu.async_copy(srcs[par].at[idxvs[par].at[j]],
                             gbufs[par].at[pl.ds(j * IDX_C, IDX_C), :],
                             sems[par])

    stage_and_fire(0, 0)
    stage_and_fire(1, 1)

    def pair_body(p, carry):
        for par in range(2):
            g = 2 * p + par
            base = wbase + g * GR
            gbuf = gbufs[par]
            sbuf = sbufs[par]

            pltpu.make_async_copy(ttable.at[pl.ds(0, GR * L), :], gbuf,
                                  sems[par]).wait()

            @pl.when(g + 2 < NG)
            def _():
                base2 = wbase + (g + 2) * GR
                ioff2 = pl.multiple_of(base2 // 2, NJ)
                pltpu.async_copy(tid2d.at[pl.ds(ioff2, NJ), :], idxvs[par],
                                 isems[par])

            @pl.when(p > 0)
            def _():
                pltpu.make_async_copy(
                    sbuf, tout.at[pl.ds(pl.multiple_of(base, GR), GR), :],
                    osems[par]).wait()

            zv = zbuf[g, :]
            denv = jnp.maximum(jnp.float32(L) - zv, 1.0)

            for r in range(GR):
                b = r * L

                def sum_body(t, acc):
                    a0, a1 = acc
                    return (a0 + gbuf[b + t, 0:16],
                            a1 + gbuf[b + t, 16:32])

                a0, a1 = lax.fori_loop(
                    0, L, sum_body,
                    (jnp.zeros((16,), jnp.float32),
                     jnp.zeros((16,), jnp.float32)),
                    unroll=5)
                nz = zv[r]
                den = denv[r]
                sbuf[r, 0:16] = (a0 - nz * r0a) / den
                sbuf[r, 16:32] = (a1 - nz * r0b) / den

            @pl.when(g + 2 < NG)
            def _():
                pltpu.make_async_copy(tid2d.at[pl.ds(0, NJ), :], idxvs[par],
                                      isems[par]).wait()
                for j in range(NJ):
                    pltpu.async_copy(srcs[par].at[idxvs[par].at[j]],
                                     gbuf.at[pl.ds(j * IDX_C, IDX_C), :],
                                     sems[par])

            pltpu.async_copy(
                sbuf, tout.at[pl.ds(pl.multiple_of(base, GR), GR), :],
                osems[par])
        return carry

    lax.fori_loop(0, NG // 2, pair_body, 0)

    for par in range(2):
        pltpu.make_async_copy(
            sbufs[par],
            tout.at[pl.ds(pl.multiple_of(wbase, GR), GR), :],
            osems[par]).wait()

    pltpu.make_async_copy(utab16.at[pl.ds(0, RPW), :], ubuf, usem).wait()
    pltpu.sync_copy(ubuf, uout.at[pl.ds(pl.multiple_of(wbase, CH), RPW), :])


@functools.partial(
    pl.kernel,
    out_type=(
        jax.ShapeDtypeStruct((B, 16), jnp.float32),
        jax.ShapeDtypeStruct((B, TOPIC_DIM), jnp.float32),
    ),
    mesh=plsc.VectorSubcoreMesh(core_axis_name="c", subcore_axis_name="s"),
    compiler_params=pltpu.CompilerParams(use_tc_tiling_on_sc=False),
    scratch_types=[
        pltpu.VMEM((NJ, IDX_C), jnp.int32),
        pltpu.VMEM((NJ, IDX_C), jnp.int32),
        pltpu.VMEM((GR * L, TOPIC_DIM), jnp.float32),
        pltpu.VMEM((GR * L, TOPIC_DIM), jnp.float32),
        pltpu.VMEM((L, CH), jnp.int32),
        pltpu.VMEM((NG, 16), jnp.float32),
        pltpu.VMEM((TOPIC_DIM,), jnp.float32),
        pltpu.VMEM((UNJ, UCHUNK), jnp.int32),
        pltpu.VMEM((RPW, 16), jnp.float32),
        pltpu.VMEM((GR, TOPIC_DIM), jnp.float32),
        pltpu.VMEM((GR, TOPIC_DIM), jnp.float32),
        pltpu.VMEM_SHARED((MAX_TOKENS, TOPIC_DIM), jnp.float32),
        pltpu.SemaphoreType.DMA,
        pltpu.SemaphoreType.DMA,
        pltpu.SemaphoreType.DMA,
        pltpu.SemaphoreType.DMA,
        pltpu.SemaphoreType.DMA,
        pltpu.SemaphoreType.DMA,
        pltpu.SemaphoreType.DMA,
        pltpu.SemaphoreType.DMA,
    ],
)
def _user_model_sc(tid2d, ids_t, ttable, uids3d, utab16, uout, tout,
                   idxv0, idxv1, gbuf0, gbuf1, cntv, zbuf, row0v, uidx, ubuf,
                   sbuf0, sbuf1, shm, sem0, sem1, osem0, osem1, usem, tsem,
                   isem0, isem1):
    _sc_body(tid2d, ids_t, ttable, uids3d, utab16, uout, tout,
             idxv0, idxv1, gbuf0, gbuf1, cntv, zbuf, row0v, uidx, ubuf,
             sbuf0, sbuf1, shm, sem0, sem1, osem0, osem1, usem, tsem,
             isem0, isem1)


def kernel(user_ids, topic_ids, user_table, topic_table):
    tid2d = topic_ids.reshape(B * L // IDX_C, IDX_C)
    ids_t = topic_ids.T
    uids3d = user_ids.reshape(NW, UNJ, UCHUNK)
    utab16 = jnp.pad(user_table, ((0, 0), (0, 1)))
    uout, tout = _user_model_sc(tid2d, ids_t, topic_table, uids3d, utab16)
    return jnp.concatenate([uout[:, :USER_DIM], tout], axis=1)


# speedup vs baseline: 1.1101x; 1.0776x over previous
import functools

import jax
import jax.numpy as jnp
from jax import lax
from jax.experimental import pallas as pl
from jax.experimental.pallas import tpu as pltpu
from jax.experimental.pallas import tpu_sc as plsc

B = 16384
L = 50
NUM_USERS = 100000
USER_DIM = 15
MAX_TOKENS = 10000
TOPIC_DIM = 32

NC = 2
NS = 16
NW = NC * NS
RPW = B // NW
CH = 128
NCH = RPW // CH
GR = 16
NG = RPW // GR
IDX_C = 100
NJ = GR * L // IDX_C
UCHUNK = 128
UNJ = RPW // UCHUNK


def _sc_body(tid2d, ids_t, ttable, uids3d, utab16,
             uout, tout,
             idxv0, idxv1, gbuf0, gbuf1, cntv, zbuf, row0v, uidx, ubuf,
             sbuf0, sbuf1, shm, sem0, sem1, osem0, osem1, usem, tsem,
             isem0, isem1, csem):
    wid = lax.axis_index("s") * NC + lax.axis_index("c")
    wbase = wid * RPW
    idxvs = (idxv0, idxv1)
    gbufs = (gbuf0, gbuf1)
    sbufs = (sbuf0, sbuf1)
    sems = (sem0, sem1)
    osems = (osem0, osem1)
    isems = (isem0, isem1)
    GBYTES = GR * L * TOPIC_DIM * 4
    OBYTES = GR * TOPIC_DIM * 4
    IBYTES = NJ * IDX_C * 4

    pltpu.sync_copy(ttable.at[0], row0v)
    r0a = row0v[0:16]
    r0b = row0v[16:32]

    pltpu.sync_copy(uids3d.at[wid], uidx)
    for j in range(UNJ):
        pltpu.async_copy(utab16.at[uidx.at[j]],
                         ubuf.at[pl.ds(j * UCHUNK, UCHUNK), :], usem)

    sid = lax.axis_index("s")

    @pl.when(sid == 0)
    def _():
        pltpu.async_copy(ttable, shm, tsem)

    pltpu.async_copy(ids_t.at[:, pl.ds(pl.multiple_of(wbase, CH), RPW)],
                     cntv, csem)

    def stage_and_fire(g, par, src):
        base = wbase + g * GR
        ioff = pl.multiple_of(base // 2, NJ)
        pltpu.sync_copy(tid2d.at[pl.ds(ioff, NJ), :], idxvs[par])
        for j in range(NJ):
            pltpu.async_copy(src.at[idxvs[par].at[j]],
                             gbufs[par].at[pl.ds(j * IDX_C, IDX_C), :],
                             sems[par])

    stage_and_fire(0, 0, ttable)
    stage_and_fire(1, 1, ttable)

    pltpu.make_async_copy(ids_t.at[:, pl.ds(0, RPW)], cntv, csem).wait()
    for c in range(NCH):
        def cnt_body(t, zs):
            return tuple(
                zs[k] + jnp.where(
                    cntv[t, c * CH + k * 16:c * CH + (k + 1) * 16] == 0,
                    1.0, 0.0)
                for k in range(CH // 16))

        zs = lax.fori_loop(0, L, cnt_body,
                           tuple(jnp.zeros((16,), jnp.float32)
                                 for _ in range(CH // 16)),
                           unroll=2)
        for k in range(CH // 16):
            zbuf[c * (CH // 16) + k, :] = zs[k]

    @pl.when(sid == 0)
    def _():
        pltpu.make_async_copy(ttable, shm, tsem).wait()

    plsc.subcore_barrier()

    def pair_body(p, carry):
        for par in range(2):
            g = 2 * p + par
            base = wbase + g * GR
            gbuf = gbufs[par]
            sbuf = sbufs[par]

            pltpu.make_async_copy(ttable.at[pl.ds(0, GR * L), :], gbuf,
                                  sems[par]).wait()

            @pl.when(g + 2 < NG)
            def _():
                base2 = wbase + (g + 2) * GR
                ioff2 = pl.multiple_of(base2 // 2, NJ)
                pltpu.async_copy(tid2d.at[pl.ds(ioff2, NJ), :], idxvs[par],
                                 isems[par])

            @pl.when(p > 0)
            def _():
                pltpu.make_async_copy(
                    sbuf, tout.at[pl.ds(pl.multiple_of(base, GR), GR), :],
                    osems[par]).wait()

            zv = zbuf[g, :]
            denv = jnp.maximum(jnp.float32(L) - zv, 1.0)

            for r in range(GR):
                b = r * L

                def sum_body(t, acc):
                    a0, a1 = acc
                    return (a0 + gbuf[b + t, 0:16],
                            a1 + gbuf[b + t, 16:32])

                a0, a1 = lax.fori_loop(
                    0, L, sum_body,
                    (jnp.zeros((16,), jnp.float32),
                     jnp.zeros((16,), jnp.float32)),
                    unroll=5)
                nz = zv[r]
                den = denv[r]
                sbuf[r, 0:16] = (a0 - nz * r0a) / den
                sbuf[r, 16:32] = (a1 - nz * r0b) / den

            @pl.when(g + 2 < NG)
            def _():
                pltpu.make_async_copy(tid2d.at[pl.ds(0, NJ), :], idxvs[par],
                                      isems[par]).wait()
                for j in range(NJ):
                    pltpu.async_copy(shm.at[idxvs[par].at[j]],
                                     gbuf.at[pl.ds(j * IDX_C, IDX_C), :],
                                     sems[par])

            pltpu.async_copy(
                sbuf, tout.at[pl.ds(pl.multiple_of(base, GR), GR), :],
                osems[par])
        return carry

    lax.fori_loop(0, NG // 2, pair_body, 0)

    for par in range(2):
        pltpu.make_async_copy(
            sbufs[par],
            tout.at[pl.ds(pl.multiple_of(wbase, GR), GR), :],
            osems[par]).wait()

    pltpu.make_async_copy(utab16.at[pl.ds(0, RPW), :], ubuf, usem).wait()
    pltpu.sync_copy(ubuf, uout.at[pl.ds(pl.multiple_of(wbase, CH), RPW), :])


@functools.partial(
    pl.kernel,
    out_type=(
        jax.ShapeDtypeStruct((B, 16), jnp.float32),
        jax.ShapeDtypeStruct((B, TOPIC_DIM), jnp.float32),
    ),
    mesh=plsc.VectorSubcoreMesh(core_axis_name="c", subcore_axis_name="s"),
    compiler_params=pltpu.CompilerParams(use_tc_tiling_on_sc=False),
    scratch_types=[
        pltpu.VMEM((NJ, IDX_C), jnp.int32),
        pltpu.VMEM((NJ, IDX_C), jnp.int32),
        pltpu.VMEM((GR * L, TOPIC_DIM), jnp.float32),
        pltpu.VMEM((GR * L, TOPIC_DIM), jnp.float32),
        pltpu.VMEM((L, RPW), jnp.int32),
        pltpu.VMEM((NG, 16), jnp.float32),
        pltpu.VMEM((TOPIC_DIM,), jnp.float32),
        pltpu.VMEM((UNJ, UCHUNK), jnp.int32),
        pltpu.VMEM((RPW, 16), jnp.float32),
        pltpu.VMEM((GR, TOPIC_DIM), jnp.float32),
        pltpu.VMEM((GR, TOPIC_DIM), jnp.float32),
        pltpu.VMEM_SHARED((MAX_TOKENS, TOPIC_DIM), jnp.float32),
        pltpu.SemaphoreType.DMA,
        pltpu.SemaphoreType.DMA,
        pltpu.SemaphoreType.DMA,
        pltpu.SemaphoreType.DMA,
        pltpu.SemaphoreType.DMA,
        pltpu.SemaphoreType.DMA,
        pltpu.SemaphoreType.DMA,
        pltpu.SemaphoreType.DMA,
        pltpu.SemaphoreType.DMA,
    ],
)
def _user_model_sc(tid2d, ids_t, ttable, uids3d, utab16, uout, tout,
                   idxv0, idxv1, gbuf0, gbuf1, cntv, zbuf, row0v, uidx, ubuf,
                   sbuf0, sbuf1, shm, sem0, sem1, osem0, osem1, usem, tsem,
                   isem0, isem1, csem):
    _sc_body(tid2d, ids_t, ttable, uids3d, utab16, uout, tout,
             idxv0, idxv1, gbuf0, gbuf1, cntv, zbuf, row0v, uidx, ubuf,
             sbuf0, sbuf1, shm, sem0, sem1, osem0, osem1, usem, tsem,
             isem0, isem1, csem)


def kernel(user_ids, topic_ids, user_table, topic_table):
    tid2d = topic_ids.reshape(B * L // IDX_C, IDX_C)
    ids_t = topic_ids.T
    uids3d = user_ids.reshape(NW, UNJ, UCHUNK)
    utab16 = jnp.pad(user_table, ((0, 0), (0, 1)))
    uout, tout = _user_model_sc(tid2d, ids_t, topic_table, uids3d, utab16)
    return jnp.concatenate([uout[:, :USER_DIM], tout], axis=1)
